# Initial kernel scaffold; baseline (speedup 1.0000x reference)
import jax
import jax.numpy as jnp
from jax.experimental import pallas as pl


def kernel(input_ids, weight):
    raise NotImplementedError("write your pallas kernel here")


# speedup vs baseline: 1.4776x; 1.4776x over previous
import functools

import jax
import jax.numpy as jnp
from jax import lax
from jax.experimental import pallas as pl
from jax.experimental.pallas import tpu as pltpu
from jax.experimental.pallas import tpu_sc as plsc

_VOCAB = 1_000_000
_EMB = 32

_B = 4096 * 200
_NC, _NS = 2, 16
_NW = _NC * _NS
_BPW = _B // _NW
_C = 1600
_NCH = _BPW // _C


def _make_sc_kernel():
    mesh = plsc.VectorSubcoreMesh(core_axis_name="c", subcore_axis_name="s")

    @functools.partial(
        pl.kernel,
        mesh=mesh,
        out_type=jax.ShapeDtypeStruct((_B, _EMB), jnp.float32),
        scratch_types=[
            pltpu.VMEM((_C,), jnp.int32),
            pltpu.VMEM((_C, _EMB), jnp.float32),
            pltpu.SemaphoreType.DMA,
        ],
        compiler_params=pltpu.CompilerParams(use_tc_tiling_on_sc=False),
    )
    def emb_kernel(idx_hbm, table_hbm, out_hbm, idx_v, rows_v, sem):
        wid = lax.axis_index("s") * _NC + lax.axis_index("c")
        base = wid * _BPW
        for g in range(_NCH):
            off = base + g * _C
            pltpu.sync_copy(idx_hbm.at[pl.ds(off, _C)], idx_v)
            pltpu.async_copy(table_hbm.at[idx_v], rows_v, sem).wait()
            pltpu.sync_copy(rows_v, out_hbm.at[pl.ds(off, _C)])

    return emb_kernel


def kernel(input_ids, weight):
    flat = input_ids.reshape(-1).astype(jnp.int32)
    out = _make_sc_kernel()(flat, weight)
    return out.reshape(input_ids.shape + (_EMB,))


# speedup vs baseline: 1.4929x; 1.0103x over previous
import functools

import jax
import jax.numpy as jnp
from jax import lax
from jax.experimental import pallas as pl
from jax.experimental.pallas import tpu as pltpu
from jax.experimental.pallas import tpu_sc as plsc

_VOCAB = 1_000_000
_EMB = 32

_B = 4096 * 200
_NC, _NS = 2, 16
_NW = _NC * _NS
_BPW = _B // _NW
_C = 1280
_NCH = _BPW // _C
_NBUF = 2


def _make_sc_kernel():
    mesh = plsc.VectorSubcoreMesh(core_axis_name="c", subcore_axis_name="s")

    @functools.partial(
        pl.kernel,
        mesh=mesh,
        out_type=jax.ShapeDtypeStruct((_B, _EMB), jnp.float32),
        scratch_types=[
            pltpu.VMEM((_BPW,), jnp.int32),
            pltpu.VMEM((_C, _EMB), jnp.float32),
            pltpu.VMEM((_C, _EMB), jnp.float32),
            pltpu.SemaphoreType.DMA,
            pltpu.SemaphoreType.DMA,
            pltpu.SemaphoreType.DMA,
            pltpu.SemaphoreType.DMA,
        ],
        compiler_params=pltpu.CompilerParams(use_tc_tiling_on_sc=False),
    )
    def emb_kernel(idx_hbm, table_hbm, out_hbm, idx_v, r0, r1, g0, g1, o0, o1):
        rows_v = [r0, r1]
        gsem = [g0, g1]
        osem = [o0, o1]
        wid = lax.axis_index("s") * _NC + lax.axis_index("c")
        base = wid * _BPW

        pltpu.sync_copy(idx_hbm.at[pl.ds(base, _BPW)], idx_v)

        gather = [None] * _NBUF
        wback = [None] * _NBUF
        for g in range(_NCH):
            b = g % _NBUF
            if wback[b] is not None:
                wback[b].wait()
            gather[b] = pltpu.async_copy(
                table_hbm.at[idx_v.at[pl.ds(g * _C, _C)]], rows_v[b], gsem[b])
            if g >= 1:
                bp = (g - 1) % _NBUF
                gather[bp].wait()
                wback[bp] = pltpu.async_copy(
                    rows_v[bp], out_hbm.at[pl.ds(base + (g - 1) * _C, _C)],
                    osem[bp])
        bl = (_NCH - 1) % _NBUF
        gather[bl].wait()
        wback[bl] = pltpu.async_copy(
            rows_v[bl], out_hbm.at[pl.ds(base + (_NCH - 1) * _C, _C)], osem[bl])
        for b in range(_NBUF):
            if wback[b] is not None:
                wback[b].wait()

    return emb_kernel


def kernel(input_ids, weight):
    flat = input_ids.reshape(-1).astype(jnp.int32)
    out = _make_sc_kernel()(flat, weight)
    return out.reshape(input_ids.shape + (_EMB,))


# speedup vs baseline: 1.4961x; 1.0021x over previous
import functools

import jax
import jax.numpy as jnp
from jax import lax
from jax.experimental import pallas as pl
from jax.experimental.pallas import tpu as pltpu
from jax.experimental.pallas import tpu_sc as plsc

_VOCAB = 1_000_000
_EMB = 32

_B = 4096 * 200
_NC, _NS = 2, 16
_NW = _NC * _NS
_BPW = _B // _NW
_C = 1600
_NCH = _BPW // _C
_NBUF = 2


def _make_sc_kernel():
    mesh = plsc.VectorSubcoreMesh(core_axis_name="c", subcore_axis_name="s")

    @functools.partial(
        pl.kernel,
        mesh=mesh,
        out_type=jax.ShapeDtypeStruct((4096, 200, _EMB), jnp.float32),
        scratch_types=[
            pltpu.VMEM((_BPW,), jnp.int32),
            pltpu.VMEM((_C, _EMB), jnp.float32),
            pltpu.VMEM((_C, _EMB), jnp.float32),
            pltpu.SemaphoreType.DMA,
            pltpu.SemaphoreType.DMA,
            pltpu.SemaphoreType.DMA,
            pltpu.SemaphoreType.DMA,
        ],
        compiler_params=pltpu.CompilerParams(use_tc_tiling_on_sc=False),
    )
    def emb_kernel(idx_hbm, table_hbm, out3_hbm, idx_v, r0, r1, g0, g1, o0, o1):
        rows_v = [r0, r1]
        gsem = [g0, g1]
        osem = [o0, o1]
        _R = _C // 200
        wid = lax.axis_index("s") * _NC + lax.axis_index("c")
        base = wid * _BPW
        row0 = wid * (_BPW // 200)

        pltpu.sync_copy(idx_hbm.at[pl.ds(base, _BPW)], idx_v)

        def wb(g, b):
            return [
                pltpu.async_copy(
                    rows_v[b].at[pl.ds(j * 200, 200)],
                    out3_hbm.at[row0 + g * _R + j], osem[b])
                for j in range(_R)
            ]

        gather = [None] * _NBUF
        wback = [None] * _NBUF
        for g in range(_NCH):
            b = g % _NBUF
            if wback[b] is not None:
                for h in wback[b]:
                    h.wait()
            gather[b] = pltpu.async_copy(
                table_hbm.at[idx_v.at[pl.ds(g * _C, _C)]], rows_v[b], gsem[b])
            if g >= 1:
                bp = (g - 1) % _NBUF
                gather[bp].wait()
                wback[bp] = wb(g - 1, bp)
        bl = (_NCH - 1) % _NBUF
        gather[bl].wait()
        wback[bl] = wb(_NCH - 1, bl)
        for b in range(_NBUF):
            if wback[b] is not None:
                for h in wback[b]:
                    h.wait()

    return emb_kernel


def kernel(input_ids, weight):
    flat = input_ids.reshape(-1).astype(jnp.int32)
    return _make_sc_kernel()(flat, weight)


# speedup vs baseline: 1.5145x; 1.0123x over previous
import functools

import jax
import jax.numpy as jnp
from jax import lax
from jax.experimental import pallas as pl
from jax.experimental.pallas import tpu as pltpu
from jax.experimental.pallas import tpu_sc as plsc

_VOCAB = 1_000_000
_EMB = 32

_B = 4096 * 200
_NC, _NS = 2, 16
_NW = _NC * _NS
_BPW = _B // _NW
_C = 1600
_NCH = _BPW // _C
_NBUF = 2


def _make_sc_kernel():
    mesh = plsc.VectorSubcoreMesh(core_axis_name="c", subcore_axis_name="s")

    @functools.partial(
        pl.kernel,
        mesh=mesh,
        out_type=jax.ShapeDtypeStruct((4096, 200, _EMB), jnp.float32),
        scratch_types=[
            pltpu.VMEM((_BPW,), jnp.int32),
            pltpu.VMEM((_C, _EMB), jnp.float32),
            pltpu.VMEM((_C, _EMB), jnp.float32),
            pltpu.SemaphoreType.DMA,
            pltpu.SemaphoreType.DMA,
            pltpu.SemaphoreType.DMA,
            pltpu.SemaphoreType.DMA,
        ],
        compiler_params=pltpu.CompilerParams(use_tc_tiling_on_sc=False),
    )
    def emb_kernel(idx_hbm, table_hbm, out3_hbm, idx_v, r0, r1, g0, g1, o0, o1):
        rows_v = [r0, r1]
        gsem = [g0, g1]
        osem = [o0, o1]
        _R = _C // 200
        wid = lax.axis_index("s") * _NC + lax.axis_index("c")
        base = wid * _BPW
        row0 = wid * (_BPW // 200)

        pltpu.sync_copy(idx_hbm.at[pl.ds(base, _BPW)], idx_v)

        def wb(g, b):
            return [
                pltpu.async_copy(
                    rows_v[b].at[pl.ds(j * 200, 200)],
                    out3_hbm.at[row0 + g * _R + j], osem[b])
                for j in range(_R)
            ]

        gather = [None] * _NBUF
        wback = [None] * _NBUF
        for g in range(_NCH):
            b = g % _NBUF
            if wback[b] is not None:
                for h in wback[b]:
                    h.wait()
            gather[b] = pltpu.async_copy(
                table_hbm.at[idx_v.at[pl.ds(g * _C, _C)]], rows_v[b], gsem[b])
            if g >= 1:
                bp = (g - 1) % _NBUF
                gather[bp].wait()
                wback[bp] = wb(g - 1, bp)
        bl = (_NCH - 1) % _NBUF
        gather[bl].wait()
        wback[bl] = wb(_NCH - 1, bl)
        for b in range(_NBUF):
            if wback[b] is not None:
                for h in wback[b]:
                    h.wait()

    return emb_kernel


def kernel(input_ids, weight):
    flat4 = input_ids.reshape(-1).astype(jnp.int32) * 4
    w4 = jnp.pad(weight, ((0, 0), (0, 3 * _EMB))).reshape(4 * _VOCAB, _EMB)
    return _make_sc_kernel()(flat4, w4)
